# baseline (speedup 1.0000x reference)
import functools

import jax
import jax.numpy as jnp
from jax import lax
from jax.experimental import pallas as pl
from jax.experimental.pallas import tpu as pltpu
from jax.experimental.pallas import tpu_sc as plsc

N_PTS = 262144
LODS_C = (16, 32, 64, 128, 256)
FD = 4
NC, NS, LANES = 2, 16, 16
NW = NC * NS
PW = N_PTS // NW
CHUNK = 128
GROUPS = CHUNK // LANES
D_IN = FD * len(LODS_C)


def _sc_interp(xs, ys, zs, fg0, fg1, fg2, fg3, fg4):
    mesh = plsc.VectorSubcoreMesh(
        core_axis_name="c", subcore_axis_name="s", num_cores=NC, num_subcores=NS
    )

    @functools.partial(
        pl.kernel,
        out_type=jax.ShapeDtypeStruct((D_IN, N_PTS), jnp.float32),
        mesh=mesh,
        compiler_params=pltpu.CompilerParams(
            needs_layout_passes=False, use_tc_tiling_on_sc=False
        ),
        scratch_types=[
            pltpu.VMEM((CHUNK,), jnp.float32),
            pltpu.VMEM((CHUNK,), jnp.float32),
            pltpu.VMEM((CHUNK,), jnp.float32),
            pltpu.VMEM((8, CHUNK), jnp.int32),
            pltpu.VMEM((8, CHUNK), jnp.float32),
            pltpu.VMEM((8 * CHUNK, FD), jnp.float32),
            pltpu.VMEM((D_IN, CHUNK), jnp.float32),
            pltpu.SemaphoreType.DMA,
        ],
    )
    def interp_kernel(xs_h, ys_h, zs_h, g0, g1, g2, g3, g4, out_h,
                      xv, yv, zv, idxb, wb, rows, feats, sem):
        grids = (g0, g1, g2, g3, g4)
        wid = lax.axis_index("s") * NC + lax.axis_index("c")
        lane = lax.iota(jnp.int32, LANES)

        def chunk_body(ci, _):
            base = wid * PW + ci * CHUNK
            pltpu.sync_copy(xs_h.at[pl.ds(base, CHUNK)], xv)
            pltpu.sync_copy(ys_h.at[pl.ds(base, CHUNK)], yv)
            pltpu.sync_copy(zs_h.at[pl.ds(base, CHUNK)], zv)

            for li, res in enumerate(LODS_C):
                grid = grids[li]
                r2 = res * res
                scale = float(res - 1)

                def idx_body(g, _):
                    g16 = g * LANES
                    x = xv[pl.ds(g16, LANES)] * scale
                    y = yv[pl.ds(g16, LANES)] * scale
                    z = zv[pl.ds(g16, LANES)] * scale
                    cx = jnp.minimum(x.astype(jnp.int32), res - 2)
                    cy = jnp.minimum(y.astype(jnp.int32), res - 2)
                    cz = jnp.minimum(z.astype(jnp.int32), res - 2)
                    fx = x - cx.astype(jnp.float32)
                    fy = y - cy.astype(jnp.float32)
                    fz = z - cz.astype(jnp.float32)
                    ibase = cx * r2 + cy * res + cz
                    one = jnp.float32(1.0)
                    wxs = (one - fx, fx)
                    wys = (one - fy, fy)
                    wzs = (one - fz, fz)
                    c = 0
                    for dx in (0, 1):
                        wyz0 = wxs[dx]
                        for dy in (0, 1):
                            wxy = wyz0 * wys[dy]
                            for dz in (0, 1):
                                idxb[c, pl.ds(g16, LANES)] = (
                                    ibase + (dx * r2 + dy * res + dz)
                                )
                                wb[c, pl.ds(g16, LANES)] = wxy * wzs[dz]
                                c += 1
                    return _

                lax.fori_loop(0, GROUPS, idx_body, None)

                cps = [
                    pltpu.async_copy(
                        grid.at[idxb.at[c]],
                        rows.at[pl.ds(c * CHUNK, CHUNK)],
                        sem,
                    )
                    for c in range(8)
                ]
                for cp in cps:
                    cp.wait()

                def comb_body(g, _):
                    g16 = g * LANES
                    ridx = g16 + lane
                    ws = [wb[c, pl.ds(g16, LANES)] for c in range(8)]
                    for f in range(FD):
                        fidx = jnp.full((LANES,), f, jnp.int32)
                        acc = ws[0] * plsc.load_gather(rows, [ridx, fidx])
                        for c in range(1, 8):
                            acc = acc + ws[c] * plsc.load_gather(
                                rows, [ridx + (c * CHUNK), fidx]
                            )
                        feats[li * FD + f, pl.ds(g16, LANES)] = acc
                    return _

                lax.fori_loop(0, GROUPS, comb_body, None)

            pltpu.sync_copy(feats, out_h.at[:, pl.ds(base, CHUNK)])
            return _

        lax.fori_loop(0, PW // CHUNK, chunk_body, None)

    return interp_kernel(xs, ys, zs, fg0, fg1, fg2, fg3, fg4)


def _mlp_body(f_ref, w0_ref, b0_ref, w1_ref, b1_ref, w2_ref, b2_ref, o_ref):
    fb = f_ref[...]
    h = lax.dot_general(
        fb, w0_ref[...], (((0,), (0,)), ((), ())),
        preferred_element_type=jnp.float32,
    )
    h = jnp.maximum(h + b0_ref[...][None, :], 0.0)
    h = jnp.maximum(
        jnp.dot(h, w1_ref[...], preferred_element_type=jnp.float32)
        + b1_ref[...][None, :],
        0.0,
    )
    o = (
        jnp.dot(h, w2_ref[...], preferred_element_type=jnp.float32)
        + b2_ref[...][None, :]
    )
    o_ref[...] = 1.0 / (1.0 + jnp.exp(-o))


def _mlp(feats, W0, b0, W1, b1, W2, b2):
    BN = 2048
    grid = (N_PTS // BN,)
    return pl.pallas_call(
        _mlp_body,
        grid=grid,
        in_specs=[
            pl.BlockSpec((D_IN, BN), lambda i: (0, i)),
            pl.BlockSpec((D_IN, 64), lambda i: (0, 0)),
            pl.BlockSpec((64,), lambda i: (0,)),
            pl.BlockSpec((64, 64), lambda i: (0, 0)),
            pl.BlockSpec((64,), lambda i: (0,)),
            pl.BlockSpec((64, 1), lambda i: (0, 0)),
            pl.BlockSpec((1,), lambda i: (0,)),
        ],
        out_specs=pl.BlockSpec((BN, 1), lambda i: (i, 0)),
        out_shape=jax.ShapeDtypeStruct((N_PTS, 1), jnp.float32),
    )(feats, W0, b0, W1, b1, W2, b2)


def kernel(x, fg0, fg1, fg2, fg3, fg4, W0, b0, W1, b1, W2, b2):
    xs, ys, zs = x[:, 0], x[:, 1], x[:, 2]
    feats = _sc_interp(xs, ys, zs, fg0, fg1, fg2, fg3, fg4)
    return _mlp(feats, W0, b0, W1, b1, W2, b2)


# speedup vs baseline: 12.2438x; 12.2438x over previous
import functools

import jax
import jax.numpy as jnp
from jax import lax
from jax.experimental import pallas as pl
from jax.experimental.pallas import tpu as pltpu
from jax.experimental.pallas import tpu_sc as plsc

N_PTS = 262144
LODS_C = (16, 32, 64, 128, 256)
FD = 4
NC, NS, LANES = 2, 16, 16
NW = NC * NS
PW = N_PTS // NW
CHUNK = 128
GROUPS = CHUNK // LANES
D_IN = FD * len(LODS_C)


def _sc_relayout(grids):
    mesh = plsc.VectorSubcoreMesh(
        core_axis_name="c", subcore_axis_name="s", num_cores=NC, num_subcores=NS
    )
    p2s = [
        g.reshape(g.shape[0] // 128, 128, 4).transpose(0, 2, 1).reshape(-1, 128)
        for g in grids
    ]
    sizes = [g.shape[0] for g in grids]
    shares = [v * 4 // 128 // NW for v in sizes]
    stages = [min(32, s) for s in shares]

    @functools.partial(
        pl.kernel,
        out_type=tuple(
            jax.ShapeDtypeStruct((v, 4), jnp.float32) for v in sizes
        ),
        mesh=mesh,
        compiler_params=pltpu.CompilerParams(
            needs_layout_passes=False, use_tc_tiling_on_sc=False
        ),
        scratch_types=[
            pltpu.VMEM((32, 128), jnp.float32),
            pltpu.VMEM((1024, 4), jnp.float32),
        ],
    )
    def relayout_kernel(g0, g1, g2, g3, g4, o0, o1, o2, o3, o4, buf, obuf):
        wid = lax.axis_index("s") * NC + lax.axis_index("c")
        lane = lax.iota(jnp.int32, LANES)
        for gi in range(5):
            p2_h = (g0, g1, g2, g3, g4)[gi]
            out_h = (o0, o1, o2, o3, o4)[gi]
            share, S = shares[gi], stages[gi]

            def stage_body(i, _, p2_h=p2_h, out_h=out_h, share=share, S=S):
                row0 = wid * share + i * S
                pltpu.sync_copy(p2_h.at[pl.ds(row0, S)], buf.at[pl.ds(0, S)])

                def blk_body(bb, _):
                    for f in range(FD):
                        colv = jnp.full((LANES,), f, jnp.int32)
                        for t in range(8):
                            v = buf[4 * bb + f, pl.ds(16 * t, 16)]
                            plsc.store_scatter(
                                obuf, [bb * 128 + (16 * t) + lane, colv], v
                            )
                    return _

                lax.fori_loop(0, S // 4, blk_body, None)
                pltpu.sync_copy(
                    obuf.at[pl.ds(0, S * 32)],
                    out_h.at[pl.ds(row0 * 32, S * 32)],
                )
                return _

            lax.fori_loop(0, share // S, stage_body, None)

    return relayout_kernel(*p2s)


def _sc_interp(xs, ys, zs, fg0, fg1, fg2, fg3, fg4):
    mesh = plsc.VectorSubcoreMesh(
        core_axis_name="c", subcore_axis_name="s", num_cores=NC, num_subcores=NS
    )

    @functools.partial(
        pl.kernel,
        out_type=jax.ShapeDtypeStruct((D_IN, N_PTS), jnp.float32),
        mesh=mesh,
        compiler_params=pltpu.CompilerParams(
            needs_layout_passes=False, use_tc_tiling_on_sc=False
        ),
        scratch_types=[
            pltpu.VMEM((CHUNK,), jnp.float32),
            pltpu.VMEM((CHUNK,), jnp.float32),
            pltpu.VMEM((CHUNK,), jnp.float32),
            pltpu.VMEM((8, CHUNK), jnp.int32),
            pltpu.VMEM((8, CHUNK), jnp.float32),
            pltpu.VMEM((8 * CHUNK, FD), jnp.float32),
            pltpu.VMEM((D_IN, CHUNK), jnp.float32),
            pltpu.SemaphoreType.DMA,
        ],
    )
    def interp_kernel(xs_h, ys_h, zs_h, g0, g1, g2, g3, g4, out_h,
                      xv, yv, zv, idxb, wb, rows, feats, sem):
        grids = (g0, g1, g2, g3, g4)
        wid = lax.axis_index("s") * NC + lax.axis_index("c")
        lane = lax.iota(jnp.int32, LANES)

        def chunk_body(ci, _):
            base = wid * PW + ci * CHUNK
            pltpu.sync_copy(xs_h.at[pl.ds(base, CHUNK)], xv)
            pltpu.sync_copy(ys_h.at[pl.ds(base, CHUNK)], yv)
            pltpu.sync_copy(zs_h.at[pl.ds(base, CHUNK)], zv)

            for li, res in enumerate(LODS_C):
                grid = grids[li]
                r2 = res * res
                scale = float(res - 1)

                def idx_body(g, _):
                    g16 = g * LANES
                    x = xv[pl.ds(g16, LANES)] * scale
                    y = yv[pl.ds(g16, LANES)] * scale
                    z = zv[pl.ds(g16, LANES)] * scale
                    cx = jnp.minimum(x.astype(jnp.int32), res - 2)
                    cy = jnp.minimum(y.astype(jnp.int32), res - 2)
                    cz = jnp.minimum(z.astype(jnp.int32), res - 2)
                    fx = x - cx.astype(jnp.float32)
                    fy = y - cy.astype(jnp.float32)
                    fz = z - cz.astype(jnp.float32)
                    ibase = cx * r2 + cy * res + cz
                    one = jnp.float32(1.0)
                    wxs = (one - fx, fx)
                    wys = (one - fy, fy)
                    wzs = (one - fz, fz)
                    c = 0
                    for dx in (0, 1):
                        wyz0 = wxs[dx]
                        for dy in (0, 1):
                            wxy = wyz0 * wys[dy]
                            for dz in (0, 1):
                                idxb[c, pl.ds(g16, LANES)] = (
                                    ibase + (dx * r2 + dy * res + dz)
                                )
                                wb[c, pl.ds(g16, LANES)] = wxy * wzs[dz]
                                c += 1
                    return _

                lax.fori_loop(0, GROUPS, idx_body, None)

                cps = [
                    pltpu.async_copy(
                        grid.at[idxb.at[c]],
                        rows.at[pl.ds(c * CHUNK, CHUNK)],
                        sem,
                    )
                    for c in range(8)
                ]
                for cp in cps:
                    cp.wait()

                def comb_body(g, _):
                    g16 = g * LANES
                    ridx = g16 + lane
                    ws = [wb[c, pl.ds(g16, LANES)] for c in range(8)]
                    for f in range(FD):
                        fidx = jnp.full((LANES,), f, jnp.int32)
                        acc = ws[0] * plsc.load_gather(rows, [ridx, fidx])
                        for c in range(1, 8):
                            acc = acc + ws[c] * plsc.load_gather(
                                rows, [ridx + (c * CHUNK), fidx]
                            )
                        feats[li * FD + f, pl.ds(g16, LANES)] = acc
                    return _

                lax.fori_loop(0, GROUPS, comb_body, None)

            pltpu.sync_copy(feats, out_h.at[:, pl.ds(base, CHUNK)])
            return _

        lax.fori_loop(0, PW // CHUNK, chunk_body, None)

    return interp_kernel(xs, ys, zs, fg0, fg1, fg2, fg3, fg4)


def _mlp_body(f_ref, w0_ref, b0_ref, w1_ref, b1_ref, w2_ref, b2_ref, o_ref):
    fb = f_ref[...]
    h = lax.dot_general(
        fb, w0_ref[...], (((0,), (0,)), ((), ())),
        preferred_element_type=jnp.float32,
    )
    h = jnp.maximum(h + b0_ref[...][None, :], 0.0)
    h = jnp.maximum(
        jnp.dot(h, w1_ref[...], preferred_element_type=jnp.float32)
        + b1_ref[...][None, :],
        0.0,
    )
    o = (
        jnp.dot(h, w2_ref[...], preferred_element_type=jnp.float32)
        + b2_ref[...][None, :]
    )
    o_ref[...] = 1.0 / (1.0 + jnp.exp(-o))


def _mlp(feats, W0, b0, W1, b1, W2, b2):
    BN = 2048
    grid = (N_PTS // BN,)
    return pl.pallas_call(
        _mlp_body,
        grid=grid,
        in_specs=[
            pl.BlockSpec((D_IN, BN), lambda i: (0, i)),
            pl.BlockSpec((D_IN, 64), lambda i: (0, 0)),
            pl.BlockSpec((64,), lambda i: (0,)),
            pl.BlockSpec((64, 64), lambda i: (0, 0)),
            pl.BlockSpec((64,), lambda i: (0,)),
            pl.BlockSpec((64, 1), lambda i: (0, 0)),
            pl.BlockSpec((1,), lambda i: (0,)),
        ],
        out_specs=pl.BlockSpec((BN, 1), lambda i: (i, 0)),
        out_shape=jax.ShapeDtypeStruct((N_PTS, 1), jnp.float32),
    )(feats, W0, b0, W1, b1, W2, b2)


def kernel(x, fg0, fg1, fg2, fg3, fg4, W0, b0, W1, b1, W2, b2):
    xs, ys, zs = x[:, 0], x[:, 1], x[:, 2]
    r0, r1, r2, r3, r4 = _sc_relayout([fg0, fg1, fg2, fg3, fg4])
    feats = _sc_interp(xs, ys, zs, r0, r1, r2, r3, r4)
    return _mlp(feats, W0, b0, W1, b1, W2, b2)


# speedup vs baseline: 17.2288x; 1.4071x over previous
import functools

import jax
import jax.numpy as jnp
from jax import lax
from jax.experimental import pallas as pl
from jax.experimental.pallas import tpu as pltpu
from jax.experimental.pallas import tpu_sc as plsc

N_PTS = 262144
LODS_C = (16, 32, 64, 128, 256)
FD = 4
NC, NS, LANES = 2, 16, 16
NW = NC * NS
PW = N_PTS // NW
CHUNK = 128
GROUPS = CHUNK // LANES
D_IN = FD * len(LODS_C)


def _sc_relayout(grids):
    mesh = plsc.VectorSubcoreMesh(
        core_axis_name="c", subcore_axis_name="s", num_cores=NC, num_subcores=NS
    )
    p2s = [
        g.reshape(g.shape[0] // 128, 128, 4).transpose(0, 2, 1).reshape(-1, 128)
        for g in grids
    ]
    sizes = [g.shape[0] for g in grids]
    shares = [v * 4 // 128 // NW for v in sizes]
    stages = [min(32, s) for s in shares]

    SBIG = 128

    @functools.partial(
        pl.kernel,
        out_type=tuple(
            jax.ShapeDtypeStruct((v, 4), jnp.float32) for v in sizes
        ),
        mesh=mesh,
        compiler_params=pltpu.CompilerParams(
            needs_layout_passes=False, use_tc_tiling_on_sc=False
        ),
        scratch_types=[
            pltpu.VMEM((2, SBIG, 128), jnp.float32),
            pltpu.VMEM((2, SBIG * 32, 4), jnp.float32),
            pltpu.SemaphoreType.DMA,
            pltpu.SemaphoreType.DMA,
        ],
    )
    def relayout_kernel(g0, g1, g2, g3, g4, o0, o1, o2, o3, o4,
                        buf2, obuf2, sem_in, sem_out):
        wid = lax.axis_index("s") * NC + lax.axis_index("c")
        lane = lax.iota(jnp.int32, LANES)

        def shuffle(buf, obuf, S, bb):
            for f in range(FD):
                colv = jnp.full((LANES,), f, jnp.int32)
                for t in range(8):
                    v = buf[4 * bb + f, pl.ds(16 * t, 16)]
                    plsc.store_scatter(
                        obuf, [bb * 128 + (16 * t) + lane, colv], v
                    )

        for gi in range(5):
            p2_h = (g0, g1, g2, g3, g4)[gi]
            out_h = (o0, o1, o2, o3, o4)[gi]
            share = shares[gi]
            row_base = wid * share

            if share < 2 * SBIG:
                S = share
                buf, obuf = buf2.at[0], obuf2.at[0]
                pltpu.sync_copy(
                    p2_h.at[pl.ds(row_base, S)], buf.at[pl.ds(0, S)]
                )

                def blk_small(bb, _, buf=buf, obuf=obuf, S=S):
                    shuffle(buf, obuf, S, bb)
                    return _

                lax.fori_loop(0, S // 4, blk_small, None)
                pltpu.sync_copy(
                    obuf.at[pl.ds(0, S * 32)],
                    out_h.at[pl.ds(row_base * 32, S * 32)],
                )
            else:
                S = SBIG
                nst = share // S

                def in_cp(i, slot, start):
                    cp = (pltpu.async_copy if start else pltpu.make_async_copy)(
                        p2_h.at[pl.ds(row_base + i * S, S)],
                        buf2.at[slot], sem_in,
                    )
                    return cp

                def out_cp(i, slot, start):
                    cp = (pltpu.async_copy if start else pltpu.make_async_copy)(
                        obuf2.at[slot],
                        out_h.at[pl.ds((row_base + i * S) * 32, S * 32)],
                        sem_out,
                    )
                    return cp

                in_cp(0, 0, True)
                in_cp(1, 1, True)

                def stage2(i2, _):
                    for slot in (0, 1):
                        i = 2 * i2 + slot
                        in_cp(i, slot, False).wait()

                        @pl.when(i2 > 0)
                        def _drain():
                            out_cp(i - 2, slot, False).wait()

                        def blk_big(bb, _, slot=slot):
                            shuffle(buf2.at[slot], obuf2.at[slot], S, bb)
                            return _

                        lax.fori_loop(0, S // 4, blk_big, None)

                        @pl.when(i + 2 < nst)
                        def _next():
                            in_cp(i + 2, slot, True)

                        out_cp(i, slot, True)
                    return _

                lax.fori_loop(0, nst // 2, stage2, None)
                out_cp(nst - 2, 0, False).wait()
                out_cp(nst - 1, 1, False).wait()

    return relayout_kernel(*p2s)


def _sc_interp(xs, ys, zs, fg0, fg1, fg2, fg3, fg4):
    mesh = plsc.VectorSubcoreMesh(
        core_axis_name="c", subcore_axis_name="s", num_cores=NC, num_subcores=NS
    )

    @functools.partial(
        pl.kernel,
        out_type=jax.ShapeDtypeStruct((D_IN, N_PTS), jnp.float32),
        mesh=mesh,
        compiler_params=pltpu.CompilerParams(
            needs_layout_passes=False, use_tc_tiling_on_sc=False
        ),
        scratch_types=[
            pltpu.VMEM((CHUNK,), jnp.float32),
            pltpu.VMEM((CHUNK,), jnp.float32),
            pltpu.VMEM((CHUNK,), jnp.float32),
            pltpu.VMEM((8, CHUNK), jnp.int32),
            pltpu.VMEM((8, CHUNK), jnp.float32),
            pltpu.VMEM((8 * CHUNK, FD), jnp.float32),
            pltpu.VMEM((D_IN, CHUNK), jnp.float32),
            pltpu.SemaphoreType.DMA,
        ],
    )
    def interp_kernel(xs_h, ys_h, zs_h, g0, g1, g2, g3, g4, out_h,
                      xv, yv, zv, idxb, wb, rows, feats, sem):
        grids = (g0, g1, g2, g3, g4)
        wid = lax.axis_index("s") * NC + lax.axis_index("c")
        lane = lax.iota(jnp.int32, LANES)

        def chunk_body(ci, _):
            base = wid * PW + ci * CHUNK
            pltpu.sync_copy(xs_h.at[pl.ds(base, CHUNK)], xv)
            pltpu.sync_copy(ys_h.at[pl.ds(base, CHUNK)], yv)
            pltpu.sync_copy(zs_h.at[pl.ds(base, CHUNK)], zv)

            for li, res in enumerate(LODS_C):
                grid = grids[li]
                r2 = res * res
                scale = float(res - 1)

                def idx_body(g, _):
                    g16 = g * LANES
                    x = xv[pl.ds(g16, LANES)] * scale
                    y = yv[pl.ds(g16, LANES)] * scale
                    z = zv[pl.ds(g16, LANES)] * scale
                    cx = jnp.minimum(x.astype(jnp.int32), res - 2)
                    cy = jnp.minimum(y.astype(jnp.int32), res - 2)
                    cz = jnp.minimum(z.astype(jnp.int32), res - 2)
                    fx = x - cx.astype(jnp.float32)
                    fy = y - cy.astype(jnp.float32)
                    fz = z - cz.astype(jnp.float32)
                    ibase = cx * r2 + cy * res + cz
                    one = jnp.float32(1.0)
                    wxs = (one - fx, fx)
                    wys = (one - fy, fy)
                    wzs = (one - fz, fz)
                    c = 0
                    for dx in (0, 1):
                        wyz0 = wxs[dx]
                        for dy in (0, 1):
                            wxy = wyz0 * wys[dy]
                            for dz in (0, 1):
                                idxb[c, pl.ds(g16, LANES)] = (
                                    ibase + (dx * r2 + dy * res + dz)
                                )
                                wb[c, pl.ds(g16, LANES)] = wxy * wzs[dz]
                                c += 1
                    return _

                lax.fori_loop(0, GROUPS, idx_body, None)

                cps = [
                    pltpu.async_copy(
                        grid.at[idxb.at[c]],
                        rows.at[pl.ds(c * CHUNK, CHUNK)],
                        sem,
                    )
                    for c in range(8)
                ]
                for cp in cps:
                    cp.wait()

                def comb_body(g, _):
                    g16 = g * LANES
                    ridx = g16 + lane
                    ws = [wb[c, pl.ds(g16, LANES)] for c in range(8)]
                    for f in range(FD):
                        fidx = jnp.full((LANES,), f, jnp.int32)
                        acc = ws[0] * plsc.load_gather(rows, [ridx, fidx])
                        for c in range(1, 8):
                            acc = acc + ws[c] * plsc.load_gather(
                                rows, [ridx + (c * CHUNK), fidx]
                            )
                        feats[li * FD + f, pl.ds(g16, LANES)] = acc
                    return _

                lax.fori_loop(0, GROUPS, comb_body, None)

            pltpu.sync_copy(feats, out_h.at[:, pl.ds(base, CHUNK)])
            return _

        lax.fori_loop(0, PW // CHUNK, chunk_body, None)

    return interp_kernel(xs, ys, zs, fg0, fg1, fg2, fg3, fg4)


def _mlp_body(f_ref, w0_ref, b0_ref, w1_ref, b1_ref, w2_ref, b2_ref, o_ref):
    fb = f_ref[...]
    h = lax.dot_general(
        fb, w0_ref[...], (((0,), (0,)), ((), ())),
        preferred_element_type=jnp.float32,
    )
    h = jnp.maximum(h + b0_ref[...][None, :], 0.0)
    h = jnp.maximum(
        jnp.dot(h, w1_ref[...], preferred_element_type=jnp.float32)
        + b1_ref[...][None, :],
        0.0,
    )
    o = (
        jnp.dot(h, w2_ref[...], preferred_element_type=jnp.float32)
        + b2_ref[...][None, :]
    )
    o_ref[...] = 1.0 / (1.0 + jnp.exp(-o))


def _mlp(feats, W0, b0, W1, b1, W2, b2):
    BN = 2048
    grid = (N_PTS // BN,)
    return pl.pallas_call(
        _mlp_body,
        grid=grid,
        in_specs=[
            pl.BlockSpec((D_IN, BN), lambda i: (0, i)),
            pl.BlockSpec((D_IN, 64), lambda i: (0, 0)),
            pl.BlockSpec((64,), lambda i: (0,)),
            pl.BlockSpec((64, 64), lambda i: (0, 0)),
            pl.BlockSpec((64,), lambda i: (0,)),
            pl.BlockSpec((64, 1), lambda i: (0, 0)),
            pl.BlockSpec((1,), lambda i: (0,)),
        ],
        out_specs=pl.BlockSpec((BN, 1), lambda i: (i, 0)),
        out_shape=jax.ShapeDtypeStruct((N_PTS, 1), jnp.float32),
    )(feats, W0, b0, W1, b1, W2, b2)


def kernel(x, fg0, fg1, fg2, fg3, fg4, W0, b0, W1, b1, W2, b2):
    xs, ys, zs = x[:, 0], x[:, 1], x[:, 2]
    r0, r1, r2, r3, r4 = _sc_relayout([fg0, fg1, fg2, fg3, fg4])
    feats = _sc_interp(xs, ys, zs, r0, r1, r2, r3, r4)
    return _mlp(feats, W0, b0, W1, b1, W2, b2)


# speedup vs baseline: 24.9775x; 1.4498x over previous
import functools

import jax
import jax.numpy as jnp
from jax import lax
from jax.experimental import pallas as pl
from jax.experimental.pallas import tpu as pltpu
from jax.experimental.pallas import tpu_sc as plsc

N_PTS = 262144
LODS_C = (16, 32, 64, 128, 256)
FD = 4
NC, NS, LANES = 2, 16, 16
NW = NC * NS
PW = N_PTS // NW
CHUNK = 128
GROUPS = CHUNK // LANES
D_IN = FD * len(LODS_C)


def _sc_relayout(grids):
    mesh = plsc.VectorSubcoreMesh(
        core_axis_name="c", subcore_axis_name="s", num_cores=NC, num_subcores=NS
    )
    p2s = [
        g.reshape(g.shape[0] // 128, 128, 4).transpose(0, 2, 1).reshape(-1, 128)
        for g in grids
    ]
    sizes = [g.shape[0] for g in grids]
    shares = [v * 4 // 128 // NW for v in sizes]
    stages = [min(32, s) for s in shares]

    SBIG = 128

    @functools.partial(
        pl.kernel,
        out_type=tuple(
            jax.ShapeDtypeStruct((v, 4), jnp.float32) for v in sizes
        ),
        mesh=mesh,
        compiler_params=pltpu.CompilerParams(
            needs_layout_passes=False, use_tc_tiling_on_sc=False
        ),
        scratch_types=[
            pltpu.VMEM((2, SBIG, 128), jnp.float32),
            pltpu.VMEM((2, SBIG * 32, 4), jnp.float32),
            pltpu.SemaphoreType.DMA,
            pltpu.SemaphoreType.DMA,
            pltpu.SemaphoreType.DMA,
            pltpu.SemaphoreType.DMA,
        ],
    )
    def relayout_kernel(g0, g1, g2, g3, g4, o0, o1, o2, o3, o4,
                        buf2, obuf2, sem_in0, sem_in1, sem_out0, sem_out1):
        sems_in = (sem_in0, sem_in1)
        sems_out = (sem_out0, sem_out1)
        wid = lax.axis_index("s") * NC + lax.axis_index("c")
        lane = lax.iota(jnp.int32, LANES)

        def shuffle(buf, obuf, S, bb):
            for f in range(FD):
                colv = jnp.full((LANES,), f, jnp.int32)
                for t in range(8):
                    v = buf[4 * bb + f, pl.ds(16 * t, 16)]
                    plsc.store_scatter(
                        obuf, [bb * 128 + (16 * t) + lane, colv], v
                    )

        for gi in range(5):
            p2_h = (g0, g1, g2, g3, g4)[gi]
            out_h = (o0, o1, o2, o3, o4)[gi]
            share = shares[gi]
            row_base = wid * share

            if share < 2 * SBIG:
                S = share
                buf, obuf = buf2.at[0], obuf2.at[0]
                pltpu.sync_copy(
                    p2_h.at[pl.ds(row_base, S)], buf.at[pl.ds(0, S)]
                )

                def blk_small(bb, _, buf=buf, obuf=obuf, S=S):
                    shuffle(buf, obuf, S, bb)
                    return _

                lax.fori_loop(0, S // 4, blk_small, None)
                pltpu.sync_copy(
                    obuf.at[pl.ds(0, S * 32)],
                    out_h.at[pl.ds(row_base * 32, S * 32)],
                )
            else:
                S = SBIG
                nst = share // S

                def in_cp(i, slot, start):
                    cp = (pltpu.async_copy if start else pltpu.make_async_copy)(
                        p2_h.at[pl.ds(row_base + i * S, S)],
                        buf2.at[slot], sems_in[slot],
                    )
                    return cp

                def out_cp(i, slot, start):
                    cp = (pltpu.async_copy if start else pltpu.make_async_copy)(
                        obuf2.at[slot],
                        out_h.at[pl.ds((row_base + i * S) * 32, S * 32)],
                        sems_out[slot],
                    )
                    return cp

                in_cp(0, 0, True)
                in_cp(1, 1, True)

                def stage2(i2, _):
                    for slot in (0, 1):
                        i = 2 * i2 + slot
                        in_cp(i, slot, False).wait()

                        @pl.when(i2 > 0)
                        def _drain():
                            out_cp(i - 2, slot, False).wait()

                        def blk_big(bb, _, slot=slot):
                            shuffle(buf2.at[slot], obuf2.at[slot], S, bb)
                            return _

                        lax.fori_loop(0, S // 4, blk_big, None)

                        @pl.when(i + 2 < nst)
                        def _next():
                            in_cp(i + 2, slot, True)

                        out_cp(i, slot, True)
                    return _

                lax.fori_loop(0, nst // 2, stage2, None)
                out_cp(nst - 2, 0, False).wait()
                out_cp(nst - 1, 1, False).wait()

    return relayout_kernel(*p2s)


def _sc_interp(xs, ys, zs, fg0, fg1, fg2, fg3, fg4):
    mesh = plsc.VectorSubcoreMesh(
        core_axis_name="c", subcore_axis_name="s", num_cores=NC, num_subcores=NS
    )

    NCH = PW // CHUNK
    NCR = 8 * len(LODS_C)

    @functools.partial(
        pl.kernel,
        out_type=jax.ShapeDtypeStruct((D_IN, N_PTS), jnp.float32),
        mesh=mesh,
        compiler_params=pltpu.CompilerParams(
            needs_layout_passes=False, use_tc_tiling_on_sc=False
        ),
        scratch_types=[
            pltpu.VMEM((PW // 2,), jnp.float32),
            pltpu.VMEM((PW // 2,), jnp.float32),
            pltpu.VMEM((PW // 2,), jnp.float32),
            pltpu.VMEM((2, NCR, CHUNK), jnp.int32),
            pltpu.VMEM((2, NCR, CHUNK), jnp.float32),
            pltpu.VMEM((2, NCR * CHUNK, FD), jnp.float32),
            pltpu.VMEM((2, D_IN, CHUNK), jnp.float32),
            pltpu.SemaphoreType.DMA,
            pltpu.SemaphoreType.DMA,
            pltpu.SemaphoreType.DMA,
            pltpu.SemaphoreType.DMA,
        ],
    )
    def interp_kernel(xs_h, ys_h, zs_h, g0, g1, g2, g3, g4, out_h,
                      xv, yv, zv, idxb, wb, rows, feats,
                      sem_g0, sem_g1, sem_f0, sem_f1):
        sems_g = (sem_g0, sem_g1)
        sems_f = (sem_f0, sem_f1)
        grids = (g0, g1, g2, g3, g4)
        wid = lax.axis_index("s") * NC + lax.axis_index("c")
        lane = lax.iota(jnp.int32, LANES)
        tbase = wid * PW
        HPW = PW // 2

        def load_window(h):
            pltpu.sync_copy(xs_h.at[pl.ds(tbase + h * HPW, HPW)], xv)
            pltpu.sync_copy(ys_h.at[pl.ds(tbase + h * HPW, HPW)], yv)
            pltpu.sync_copy(zs_h.at[pl.ds(tbase + h * HPW, HPW)], zv)

        load_window(0)

        def prep(ci, slot):
            for li, res in enumerate(LODS_C):
                r2 = res * res
                scale = float(res - 1)

                def idx_body(g, _, li=li, res=res, r2=r2, scale=scale):
                    p0 = ((ci * CHUNK) & (PW // 2 - 1)) + g * LANES
                    g16 = g * LANES
                    x = xv[pl.ds(p0, LANES)] * scale
                    y = yv[pl.ds(p0, LANES)] * scale
                    z = zv[pl.ds(p0, LANES)] * scale
                    cx = jnp.minimum(x.astype(jnp.int32), res - 2)
                    cy = jnp.minimum(y.astype(jnp.int32), res - 2)
                    cz = jnp.minimum(z.astype(jnp.int32), res - 2)
                    fx = x - cx.astype(jnp.float32)
                    fy = y - cy.astype(jnp.float32)
                    fz = z - cz.astype(jnp.float32)
                    ibase = cx * r2 + cy * res + cz
                    one = jnp.float32(1.0)
                    wxs = (one - fx, fx)
                    wys = (one - fy, fy)
                    wzs = (one - fz, fz)
                    c = 0
                    for dx in (0, 1):
                        for dy in (0, 1):
                            wxy = wxs[dx] * wys[dy]
                            for dz in (0, 1):
                                idxb[slot, li * 8 + c, pl.ds(g16, LANES)] = (
                                    ibase + (dx * r2 + dy * res + dz)
                                )
                                wb[slot, li * 8 + c, pl.ds(g16, LANES)] = (
                                    wxy * wzs[dz]
                                )
                                c += 1
                    return _

                lax.fori_loop(0, GROUPS, idx_body, None)
            for li in range(len(LODS_C)):
                for c in range(8):
                    k = li * 8 + c
                    pltpu.async_copy(
                        grids[li].at[idxb.at[slot, k]],
                        rows.at[slot, pl.ds(k * CHUNK, CHUNK)],
                        sems_g[slot],
                    )

        def wait_gathers(slot):
            for li in range(len(LODS_C)):
                for c in range(8):
                    k = li * 8 + c
                    pltpu.make_async_copy(
                        grids[li].at[idxb.at[slot, k]],
                        rows.at[slot, pl.ds(k * CHUNK, CHUNK)],
                        sems_g[slot],
                    ).wait()

        def feats_cp(ci, slot, start):
            return (pltpu.async_copy if start else pltpu.make_async_copy)(
                feats.at[slot],
                out_h.at[:, pl.ds(tbase + ci * CHUNK, CHUNK)],
                sems_f[slot],
            )

        def combine(ci, slot):
            rows2 = rows.at[slot]
            for li in range(len(LODS_C)):

                def comb_body(g, _, li=li):
                    g16 = g * LANES
                    ridx = g16 + lane
                    ws = [
                        wb[slot, li * 8 + c, pl.ds(g16, LANES)]
                        for c in range(8)
                    ]
                    for f in range(FD):
                        fidx = jnp.full((LANES,), f, jnp.int32)
                        acc = ws[0] * plsc.load_gather(
                            rows2, [ridx + (li * 8 * CHUNK), fidx]
                        )
                        for c in range(1, 8):
                            acc = acc + ws[c] * plsc.load_gather(
                                rows2, [ridx + ((li * 8 + c) * CHUNK), fidx]
                            )
                        feats[slot, li * FD + f, pl.ds(g16, LANES)] = acc
                    return _

                lax.fori_loop(0, GROUPS, comb_body, None)

        prep(0, 0)

        HALF_SWITCH = (NCH // 2 - 2) // 2

        def pair_body(i2, _):
            for sub in (0, 1):
                i = 2 * i2 + sub
                if sub == 1:
                    @pl.when(i2 == HALF_SWITCH)
                    def _reload():
                        load_window(1)

                prep(i + 1, 1 - sub)
                wait_gathers(sub)

                @pl.when(i2 > 0)
                def _drain(i=i, sub=sub):
                    feats_cp(i - 2, sub, False).wait()

                combine(i, sub)
                feats_cp(i, sub, True)
            return _

        lax.fori_loop(0, (NCH - 2) // 2, pair_body, None)
        for i in (NCH - 2, NCH - 1):
            sub = i % 2
            if i + 1 < NCH:
                prep(i + 1, 1 - sub)
            wait_gathers(sub)
            feats_cp(i - 2, sub, False).wait()
            combine(i, sub)
            feats_cp(i, sub, True)
        feats_cp(NCH - 2, 0, False).wait()
        feats_cp(NCH - 1, 1, False).wait()

    return interp_kernel(xs, ys, zs, fg0, fg1, fg2, fg3, fg4)


def _mlp_body(f_ref, w0_ref, b0_ref, w1_ref, b1_ref, w2_ref, b2_ref, o_ref):
    fb = f_ref[...]
    h = lax.dot_general(
        fb, w0_ref[...], (((0,), (0,)), ((), ())),
        preferred_element_type=jnp.float32,
    )
    h = jnp.maximum(h + b0_ref[...][None, :], 0.0)
    h = jnp.maximum(
        jnp.dot(h, w1_ref[...], preferred_element_type=jnp.float32)
        + b1_ref[...][None, :],
        0.0,
    )
    o = (
        jnp.dot(h, w2_ref[...], preferred_element_type=jnp.float32)
        + b2_ref[...][None, :]
    )
    o_ref[...] = 1.0 / (1.0 + jnp.exp(-o))


def _mlp(feats, W0, b0, W1, b1, W2, b2):
    BN = 2048
    grid = (N_PTS // BN,)
    return pl.pallas_call(
        _mlp_body,
        grid=grid,
        in_specs=[
            pl.BlockSpec((D_IN, BN), lambda i: (0, i)),
            pl.BlockSpec((D_IN, 64), lambda i: (0, 0)),
            pl.BlockSpec((64,), lambda i: (0,)),
            pl.BlockSpec((64, 64), lambda i: (0, 0)),
            pl.BlockSpec((64,), lambda i: (0,)),
            pl.BlockSpec((64, 1), lambda i: (0, 0)),
            pl.BlockSpec((1,), lambda i: (0,)),
        ],
        out_specs=pl.BlockSpec((BN, 1), lambda i: (i, 0)),
        out_shape=jax.ShapeDtypeStruct((N_PTS, 1), jnp.float32),
    )(feats, W0, b0, W1, b1, W2, b2)


def kernel(x, fg0, fg1, fg2, fg3, fg4, W0, b0, W1, b1, W2, b2):
    xs, ys, zs = x[:, 0], x[:, 1], x[:, 2]
    r0, r1, r2, r3, r4 = _sc_relayout([fg0, fg1, fg2, fg3, fg4])
    feats = _sc_interp(xs, ys, zs, r0, r1, r2, r3, r4)
    return _mlp(feats, W0, b0, W1, b1, W2, b2)
